# baseline (speedup 1.0000x reference)
import functools

import jax
import jax.numpy as jnp
from jax import lax
from jax.experimental import pallas as pl
from jax.experimental.pallas import tpu as pltpu
from jax.experimental.pallas import tpu_sc as plsc

_NC = 2
_NS = 16
_NW = _NC * _NS
_LANES = 16
_NCHUNK = 16
_BB = 32


def _sc_gather(table_flat, ids_flat, base_row, n_rows, Hn, V, D):
    per_w = n_rows // _NW
    CH = 128
    n_iter = per_w // CH

    mesh = plsc.VectorSubcoreMesh(
        core_axis_name="c", subcore_axis_name="s",
        num_cores=_NC, num_subcores=_NS,
    )

    @functools.partial(
        pl.kernel,
        out_type=jax.ShapeDtypeStruct((n_rows, D), jnp.float32),
        mesh=mesh,
        scratch_types=[
            pltpu.VMEM((CH,), jnp.int32),
            pltpu.VMEM((CH, D), jnp.float32),
            pltpu.SemaphoreType.DMA,
        ],
    )
    def k(ids_hbm, table_hbm, out_hbm, idx_v, rows_v, sem):
        wid = lax.axis_index("s") * _NC + lax.axis_index("c")
        base = wid * per_w

        def body(i, carry):
            start = base + i * CH
            pltpu.sync_copy(ids_hbm.at[pl.ds(base_row + start, CH)], idx_v)

            def off_body(j, carry2):
                p = lax.iota(jnp.int32, 16) + (base_row + start + j * _LANES)
                h = lax.rem(p, Hn)
                sl = pl.ds(j * _LANES, _LANES)
                idx_v[sl] = idx_v[sl] + h * V
                return carry2

            lax.fori_loop(0, CH // _LANES, off_body, 0, unroll=True)
            pltpu.async_copy(table_hbm.at[idx_v], rows_v, sem).wait()
            pltpu.sync_copy(rows_v, out_hbm.at[pl.ds(start, CH)])
            return carry

        lax.fori_loop(0, n_iter, body, 0)

    return k(ids_flat, table_flat)


def _tc_post(gathered, strand_ids3, strand_table, pos_table, ln_gamma,
             ln_beta, out_prev, chunk, nchunk):
    Bc, Hn, D = gathered.shape
    B = Bc * nchunk
    grid = (Bc // _BB,)
    goff = chunk * (Bc // _BB)

    have_prev = out_prev is not None

    def body(g_ref, s_ref, st_ref, pt_ref, gm_ref, bt_ref, *rest):
        o_ref = rest[-1]
        x = g_ref[...].reshape(_BB, Hn, D)
        s = s_ref[0, 0, :].astype(jnp.float32)
        st = st_ref[...]
        semb = st[0][None, :] + s[:, None] * (st[1] - st[0])[None, :]
        x = x + pt_ref[...][None, :, :] + semb[:, None, :]
        mean = jnp.mean(x, axis=-1, keepdims=True)
        xc = x - mean
        var = jnp.mean(xc * xc, axis=-1, keepdims=True)
        y = xc * lax.rsqrt(var + 1e-5)
        o_ref[...] = y * gm_ref[...][None, None, :] + bt_ref[...][None, None, :]

    in_specs = [
        pl.BlockSpec((_BB * Hn, D), lambda i: (i, 0)),
        pl.BlockSpec((1, 1, _BB), lambda i: (i, 0, 0)),
        pl.BlockSpec((2, D), lambda i: (0, 0)),
        pl.BlockSpec((Hn, D), lambda i: (0, 0)),
        pl.BlockSpec((D,), lambda i: (0,)),
        pl.BlockSpec((D,), lambda i: (0,)),
    ]
    args = [gathered.reshape(Bc * Hn, D), strand_ids3, strand_table, pos_table,
            ln_gamma, ln_beta]
    if have_prev:
        in_specs.append(pl.BlockSpec(memory_space=pl.ANY))
        args.append(out_prev)
    return pl.pallas_call(
        body,
        grid=grid,
        in_specs=in_specs,
        out_specs=pl.BlockSpec((_BB, Hn, D), lambda i: (i + goff, 0, 0)),
        out_shape=jax.ShapeDtypeStruct((B, Hn, D), jnp.float32),
        input_output_aliases={6: 0} if have_prev else {},
    )(*args)


def kernel(cluster_ids, strand_ids, cluster_tables, strand_table, pos_table,
           ln_gamma, ln_beta):
    B, Hn = cluster_ids.shape
    _, V, D = cluster_tables.shape
    table_flat = cluster_tables.reshape(Hn * V, D)
    ids_flat = cluster_ids.reshape(B * Hn)
    Bc = B // _NCHUNK
    rows_c = Bc * Hn
    out = None
    for c in range(_NCHUNK):
        g = _sc_gather(table_flat, ids_flat, c * rows_c, rows_c, Hn, V, D)
        s3 = lax.dynamic_slice_in_dim(strand_ids, c * Bc, Bc).reshape(
            Bc // _BB, 1, _BB)
        out = _tc_post(g.reshape(Bc, Hn, D), s3, strand_table,
                       pos_table, ln_gamma, ln_beta, out, c, _NCHUNK)
    return out


# speedup vs baseline: 1.0034x; 1.0034x over previous
import functools

import jax
import jax.numpy as jnp
from jax import lax
from jax.experimental import pallas as pl
from jax.experimental.pallas import tpu as pltpu
from jax.experimental.pallas import tpu_sc as plsc

_NC = 2
_NS = 16
_NW = _NC * _NS
_LANES = 16
_NCHUNK = 4
_BB = 32


def _sc_gather(table_flat, ids_flat, base_row, n_rows, Hn, V, D):
    per_w = n_rows // _NW
    CH = 128
    n_iter = per_w // CH
    assert per_w % CH == 0 and n_rows % _NW == 0

    mesh = plsc.VectorSubcoreMesh(
        core_axis_name="c", subcore_axis_name="s",
        num_cores=_NC, num_subcores=_NS,
    )

    @functools.partial(
        pl.kernel,
        out_type=jax.ShapeDtypeStruct((n_rows, D), jnp.float32),
        mesh=mesh,
        scratch_types=[
            pltpu.VMEM((CH,), jnp.int32),
            pltpu.VMEM((CH, D), jnp.float32),
            pltpu.SemaphoreType.DMA,
        ],
    )
    def k(ids_hbm, table_hbm, out_hbm, idx_v, rows_v, sem):
        wid = lax.axis_index("s") * _NC + lax.axis_index("c")
        base = wid * per_w

        def body(i, carry):
            start = base + i * CH
            pltpu.sync_copy(ids_hbm.at[pl.ds(base_row + start, CH)], idx_v)

            def off_body(j, carry2):
                p = lax.iota(jnp.int32, 16) + (base_row + start + j * _LANES)
                h = lax.rem(p, Hn)
                sl = pl.ds(j * _LANES, _LANES)
                idx_v[sl] = idx_v[sl] + h * V
                return carry2

            lax.fori_loop(0, CH // _LANES, off_body, 0, unroll=True)
            pltpu.async_copy(table_hbm.at[idx_v], rows_v, sem).wait()
            pltpu.sync_copy(rows_v, out_hbm.at[pl.ds(start, CH)])
            return carry

        lax.fori_loop(0, n_iter, body, 0)

    return k(ids_flat, table_flat)


def _tc_post(gathered, strand_ids3, strand_table, pos_table, ln_gamma,
             ln_beta, out_prev, chunk, nchunk):
    Bc, Hn, D = gathered.shape
    B = Bc * nchunk
    grid = (Bc // _BB,)
    goff = chunk * (Bc // _BB)

    have_prev = out_prev is not None

    def body(g_ref, s_ref, st_ref, pt_ref, gm_ref, bt_ref, *rest):
        o_ref = rest[-1]
        x = g_ref[...].reshape(_BB, Hn, D)
        s = s_ref[0, 0, :].astype(jnp.float32)
        st = st_ref[...]
        semb = st[0][None, :] + s[:, None] * (st[1] - st[0])[None, :]
        x = x + pt_ref[...][None, :, :] + semb[:, None, :]
        mean = jnp.mean(x, axis=-1, keepdims=True)
        xc = x - mean
        var = jnp.mean(xc * xc, axis=-1, keepdims=True)
        y = xc * lax.rsqrt(var + 1e-5)
        o_ref[...] = y * gm_ref[...][None, None, :] + bt_ref[...][None, None, :]

    in_specs = [
        pl.BlockSpec((_BB * Hn, D), lambda i: (i, 0)),
        pl.BlockSpec((1, 1, _BB), lambda i: (i, 0, 0)),
        pl.BlockSpec((2, D), lambda i: (0, 0)),
        pl.BlockSpec((Hn, D), lambda i: (0, 0)),
        pl.BlockSpec((D,), lambda i: (0,)),
        pl.BlockSpec((D,), lambda i: (0,)),
    ]
    args = [gathered.reshape(Bc * Hn, D), strand_ids3, strand_table, pos_table,
            ln_gamma, ln_beta]
    if have_prev:
        in_specs.append(pl.BlockSpec(memory_space=pl.ANY))
        args.append(out_prev)
    return pl.pallas_call(
        body,
        grid=grid,
        in_specs=in_specs,
        out_specs=pl.BlockSpec((_BB, Hn, D), lambda i: (i + goff, 0, 0)),
        out_shape=jax.ShapeDtypeStruct((B, Hn, D), jnp.float32),
        input_output_aliases={6: 0} if have_prev else {},
    )(*args)


def kernel(cluster_ids, strand_ids, cluster_tables, strand_table, pos_table,
           ln_gamma, ln_beta):
    B, Hn = cluster_ids.shape
    _, V, D = cluster_tables.shape
    table_flat = cluster_tables.reshape(Hn * V, D)
    ids_flat = cluster_ids.reshape(B * Hn)
    Bc = B // _NCHUNK
    rows_c = Bc * Hn
    out = None
    for c in range(_NCHUNK):
        g = _sc_gather(table_flat, ids_flat, c * rows_c, rows_c, Hn, V, D)
        s3 = lax.dynamic_slice_in_dim(strand_ids, c * Bc, Bc).reshape(
            Bc // _BB, 1, _BB)
        out = _tc_post(g.reshape(Bc, Hn, D), s3, strand_table,
                       pos_table, ln_gamma, ln_beta, out, c, _NCHUNK)
    return out
